# baseline (speedup 1.0000x reference)
import jax
import jax.numpy as jnp
from jax import lax
from jax.experimental import pallas as pl
from jax.experimental.pallas import tpu as pltpu
from jax.experimental.pallas import tpu_sc as plsc

NC = 2
NS = 16
NW = NC * NS
BW = 512
NBB = BW // 128


def _body(x5, table, out5, i0, i1, s0, s1, t0, t1, gs0, gs1, ws0, ws1):
    idxs = (i0, i1)
    stag = (s0, s1)
    tbuf = (t0, t1)
    gsem = (gs0, gs1)
    wsem = (ws0, ws1)
    H = out5.shape[0]
    D = table.shape[1]
    NDB = D // 8
    c = lax.axis_index("c")
    s = lax.axis_index("s")
    wid = s * NC + c
    bb0 = wid * NBB

    iota = jax.lax.iota(jnp.int32, 16)
    rowvecs = [[bbl * 128 + bc0 * 16 + iota for bc0 in range(8)]
               for bbl in range(NBB)]

    def load_dec_fire(h, p):
        hb = lax.shift_right_logical(h, 3)
        hr = lax.bitwise_and(h, 7)
        pltpu.sync_copy(x5.at[hb, pl.ds(bb0, NBB), pl.ds(hr, 1), :], idxs[p])
        for i in range(NBB):
            for j in range(8):
                sl = (i, 0, pl.ds(j * 16, 16))
                idxs[p][sl] = idxs[p][sl] - 1
        for i in range(NBB):
            pltpu.async_copy(
                table.at[idxs[p].at[i, 0]], stag[p].at[pl.ds(i * 128, 128)], gsem[p]
            )

    def wait_gather(p):
        pltpu.make_async_copy(table.at[pl.ds(0, BW)], stag[p], gsem[p]).wait()

    def transpose(p):
        for db in range(NDB):
            for dr in range(8):
                col = jnp.full((16,), db * 8 + dr, jnp.int32)
                for bbl in range(NBB):
                    for bc0 in range(8):
                        v = plsc.load_gather(stag[p], [rowvecs[bbl][bc0], col])
                        tbuf[p][db, bbl, dr, pl.ds(bc0 * 16, 16)] = v

    def fire_wb(h, p):
        pltpu.async_copy(tbuf[p], out5.at[h, :, pl.ds(bb0, NBB), :, :], wsem[p])

    def wait_wb(h, p):
        pltpu.make_async_copy(tbuf[p], out5.at[h, :, pl.ds(bb0, NBB), :, :],
                              wsem[p]).wait()

    def _one(h, p, q):
        @pl.when(h < H - 1)
        def _():
            load_dec_fire(h + 1, q)
        wait_gather(p)

        @pl.when(h >= 2)
        def _():
            wait_wb(h - 2, p)
        transpose(p)
        fire_wb(h, p)

    load_dec_fire(jnp.int32(0), 0)

    @pl.loop(0, H, step=2)
    def step(h):
        _one(h, 0, 1)
        _one(h + 1, 1, 0)

    wait_wb(H - 2, 0)
    wait_wb(H - 1, 1)


def kernel(x, table):
    B, H = x.shape
    V, D = table.shape
    x5 = x.T.reshape(H // 8, 8, B // 128, 128).transpose(0, 2, 1, 3)
    mesh = plsc.VectorSubcoreMesh(core_axis_name="c", subcore_axis_name="s")
    run = pl.kernel(
        _body,
        out_type=jax.ShapeDtypeStruct((H, D // 8, B // 128, 8, 128), jnp.float32),
        mesh=mesh,
        scratch_types=[
            pltpu.VMEM((NBB, 1, 128), jnp.int32),
            pltpu.VMEM((NBB, 1, 128), jnp.int32),
            pltpu.VMEM((BW, D), jnp.float32),
            pltpu.VMEM((BW, D), jnp.float32),
            pltpu.VMEM((D // 8, NBB, 8, 128), jnp.float32),
            pltpu.VMEM((D // 8, NBB, 8, 128), jnp.float32),
            pltpu.SemaphoreType.DMA,
            pltpu.SemaphoreType.DMA,
            pltpu.SemaphoreType.DMA,
            pltpu.SemaphoreType.DMA,
        ],
        compiler_params=pltpu.CompilerParams(
            use_tc_tiling_on_sc=False, needs_layout_passes=False
        ),
    )
    out5 = run(x5, table)
    return out5.transpose(2, 4, 0, 1, 3).reshape(B, H, D)
